# baseline (speedup 1.0000x reference)
import functools

import jax
import jax.numpy as jnp
from jax import lax
from jax.experimental import pallas as pl
from jax.experimental.pallas import tpu as pltpu
from jax.experimental.pallas import tpu_sc as plsc

N = 16384
IDIM = 128
ODIM = 1000
K = 64
KP1 = K + 1
TW = 2 * IDIM

_info = plsc.get_sparse_core_info()
_NC, _NS, _L = _info.num_cores, _info.num_subcores, _info.num_lanes
_NW = _NC * _NS
_CHUNK = N // _NW
_STEP = 128
_NSTEPS = _CHUNK // _STEP
_NR = N // IDIM


def _sc_body(x_hbm, tgt_hbm, tab_hbm, noise_hbm,
             pmt_hbm, wn_hbm,
             xv0, xv1, wv0, wv1, idx0, idx1, pmt_v, nidx, wnv,
             sx0, sx1, sw0, sw1, sem):
    cid = lax.axis_index("c")
    sid = lax.axis_index("s")
    wid = sid * _NC + cid
    base = wid * _CHUNK

    xvs, wvs, idxs = (xv0, xv1), (wv0, wv1), (idx0, idx1)
    sxs, sws = (sx0, sx1), (sw0, sw1)

    lane = lax.iota(jnp.int32, _L)
    perms = [lane ^ 1, lane ^ 2, lane ^ 4, lane ^ 8]
    masks = [(lane % (2 * m)) < m for m in (1, 2, 4, 8)]

    def start(step, buf):
        rb = base + step * _STEP
        pltpu.sync_copy(tgt_hbm.at[pl.ds(rb, _STEP)], idxs[buf])
        pltpu.async_copy(x_hbm.at[pl.ds(rb, _STEP)], xvs[buf], sxs[buf])
        pltpu.async_copy(tab_hbm.at[idxs[buf]], wvs[buf], sws[buf])

    def wait_and_compute(s, buf):
        xv, wv = xvs[buf], wvs[buf]
        pltpu.make_async_copy(
            x_hbm.at[pl.ds(0, _STEP)], xv, sxs[buf]).wait()
        pltpu.make_async_copy(
            tab_hbm.at[idxs[buf]], wv, sws[buf]).wait()

        def group(hg, prev):
            stack = []
            for u in range(8):
                r = hg * 8 + u
                c1 = xv[r, pl.ds(0, _L)] * wv[r, pl.ds(0, _L)]
                c2 = xv[r, pl.ds(4 * _L, _L)] * wv[r, pl.ds(4 * _L, _L)]
                for q in range(1, 4):
                    c1 = c1 + xv[r, pl.ds(q * _L, _L)] * wv[r, pl.ds(q * _L, _L)]
                    c2 = c2 + (xv[r, pl.ds((q + 4) * _L, _L)] *
                               wv[r, pl.ds((q + 4) * _L, _L)])
                v = c1 + c2 + wv[r, pl.ds(IDIM, _L)]
                lvl = 0
                while stack and stack[-1][0] == lvl:
                    _lvl, a = stack.pop()
                    mk, pm = masks[lvl], perms[lvl]
                    v = jnp.where(mk, a + jnp.take(a, pm),
                                  v + jnp.take(v, pm))
                    lvl += 1
                stack.append((lvl, v))
            half = stack[0][1]
            mk, pm = masks[3], perms[3]
            full = jnp.where(mk, prev + jnp.take(prev, pm),
                             half + jnp.take(half, pm))

            @pl.when(hg % 2 == 1)
            def _():
                pmt_v[pl.ds(s * _STEP + (hg - 1) * 8, _L)] = full

            return half

        lax.fori_loop(0, _STEP // 8, group,
                      jnp.zeros((_L,), jnp.float32))

    start(0, 0)

    def ring(g, _):
        s0 = 2 * g
        start(s0 + 1, 1)
        wait_and_compute(s0, 0)

        @pl.when(s0 + 2 < _NSTEPS)
        def _():
            start(s0 + 2, 0)

        wait_and_compute(s0 + 1, 1)
        return 0

    lax.fori_loop(0, _NSTEPS // 2, ring, 0)

    for s in range(_CHUNK // IDIM):
        pltpu.sync_copy(pmt_v.at[pl.ds(s * IDIM, IDIM)],
                        pmt_hbm.at[base // IDIM + s])

    @pl.when(wid == 0)
    def _():
        pltpu.sync_copy(noise_hbm, nidx)
        pltpu.async_copy(tab_hbm.at[nidx], wnv, sem).wait()
        pltpu.sync_copy(wnv, wn_hbm)


_sc_call = functools.partial(
    pl.kernel,
    mesh=plsc.VectorSubcoreMesh(core_axis_name="c", subcore_axis_name="s"),
    out_type=[
        jax.ShapeDtypeStruct((_NR, IDIM), jnp.float32),
        jax.ShapeDtypeStruct((K, TW), jnp.float32),
    ],
    scratch_types=[
        pltpu.VMEM((_STEP, IDIM), jnp.float32),
        pltpu.VMEM((_STEP, IDIM), jnp.float32),
        pltpu.VMEM((_STEP, TW), jnp.float32),
        pltpu.VMEM((_STEP, TW), jnp.float32),
        pltpu.VMEM((_STEP,), jnp.int32),
        pltpu.VMEM((_STEP,), jnp.int32),
        pltpu.VMEM((_CHUNK,), jnp.float32),
        pltpu.VMEM((K,), jnp.int32),
        pltpu.VMEM((K, TW), jnp.float32),
        pltpu.SemaphoreType.DMA,
        pltpu.SemaphoreType.DMA,
        pltpu.SemaphoreType.DMA,
        pltpu.SemaphoreType.DMA,
        pltpu.SemaphoreType.DMA,
    ],
)(_sc_body)


_B = 1024


def _tc_body(x_ref, pmt_ref, wt_ref, cpad_ref, logits_ref):
    m = lax.dot_general(x_ref[...], wt_ref[...],
                        dimension_numbers=(((1,), (0,)), ((), ())),
                        preferred_element_type=jnp.float32)
    col = lax.broadcasted_iota(jnp.int32, (_B, KP1), 1)
    row_i = lax.broadcasted_iota(jnp.int32, (IDIM, IDIM), 0)
    col_i = lax.broadcasted_iota(jnp.int32, (IDIM, IDIM), 1)
    ident = (row_i == col_i).astype(jnp.float32)
    pm = jnp.concatenate(
        [lax.dot_general(ident, pmt_ref[pl.ds(c, 1), :],
                         dimension_numbers=(((1,), (1,)), ((), ())),
                         preferred_element_type=jnp.float32)
         for c in range(_B // IDIM)], axis=0)
    logits_ref[...] = jnp.where(col == 0, pm, -(m + cpad_ref[...]))


_tc_call = pl.pallas_call(
    _tc_body,
    grid=(N // _B,),
    in_specs=[
        pl.BlockSpec((_B, IDIM), lambda i: (i, 0)),
        pl.BlockSpec((_B // IDIM, IDIM), lambda i: (i, 0)),
        pl.BlockSpec((IDIM, KP1), lambda i: (0, 0)),
        pl.BlockSpec((1, KP1), lambda i: (0, 0)),
    ],
    out_specs=pl.BlockSpec((_B, KP1), lambda i: (i, 0)),
    out_shape=jax.ShapeDtypeStruct((N, KP1), jnp.float32),
)


import numpy as _np

_NOISE = _np.asarray(
    jax.random.randint(jax.random.key(42), (K,), 0, ODIM), dtype=_np.int32)


def kernel(input, target, weight, bias, unigram_prob):
    noise = jnp.asarray(_NOISE)
    cvec = bias - jnp.log(K * unigram_prob)
    table = jnp.concatenate(
        [weight, cvec[:, None],
         jnp.zeros((ODIM, TW - IDIM - 1), jnp.float32)], axis=1)

    pmt, wn2 = _sc_call(input, target.astype(jnp.int32), table,
                        noise.astype(jnp.int32))

    wt_pad = jnp.concatenate(
        [jnp.zeros((IDIM, 1), jnp.float32), wn2[:, :IDIM].T], axis=1)
    cn_pad = jnp.concatenate(
        [jnp.zeros((1,), jnp.float32), wn2[:, IDIM]]).reshape(1, KP1)

    logits = _tc_call(input, pmt, wt_pad, cn_pad)
    nce_target = jnp.ones((N, KP1), jnp.float32)
    return (logits, nce_target)


# speedup vs baseline: 1.0681x; 1.0681x over previous
import functools

import jax
import jax.numpy as jnp
from jax import lax
from jax.experimental import pallas as pl
from jax.experimental.pallas import tpu as pltpu
from jax.experimental.pallas import tpu_sc as plsc

N = 16384
IDIM = 128
ODIM = 1000
K = 64
KP1 = K + 1
TW = IDIM + 16

_info = plsc.get_sparse_core_info()
_NC, _NS, _L = _info.num_cores, _info.num_subcores, _info.num_lanes
_NW = _NC * _NS
_CHUNK = N // _NW
_STEP = 128
_NSTEPS = _CHUNK // _STEP
_NR = N // IDIM


def _sc_body(x_hbm, tgt_hbm, tab_hbm, noise_hbm,
             pmt_hbm, wn_hbm,
             xv0, xv1, wv0, wv1, idx0, idx1, pmt_v, nidx, wnv,
             sx0, sx1, sw0, sw1, sem):
    cid = lax.axis_index("c")
    sid = lax.axis_index("s")
    wid = sid * _NC + cid
    base = wid * _CHUNK

    xvs, wvs, idxs = (xv0, xv1), (wv0, wv1), (idx0, idx1)
    sxs, sws = (sx0, sx1), (sw0, sw1)

    lane = lax.iota(jnp.int32, _L)
    perms = [lane ^ 1, lane ^ 2, lane ^ 4, lane ^ 8]
    masks = [(lane % (2 * m)) < m for m in (1, 2, 4, 8)]

    def start(step, buf):
        rb = base + step * _STEP
        pltpu.sync_copy(tgt_hbm.at[pl.ds(rb, _STEP)], idxs[buf])
        pltpu.async_copy(x_hbm.at[pl.ds(rb, _STEP)], xvs[buf], sxs[buf])
        pltpu.async_copy(tab_hbm.at[idxs[buf]], wvs[buf], sws[buf])

    def wait_and_compute(s, buf):
        xv, wv = xvs[buf], wvs[buf]
        pltpu.make_async_copy(
            x_hbm.at[pl.ds(0, _STEP)], xv, sxs[buf]).wait()
        pltpu.make_async_copy(
            tab_hbm.at[idxs[buf]], wv, sws[buf]).wait()

        def group(hg, prev):
            stack = []
            for u in range(8):
                r = hg * 8 + u
                p = [xv[r, pl.ds(q * _L, _L)] * wv[r, pl.ds(q * _L, _L)]
                     for q in range(IDIM // _L)]
                t = [p[0] + p[1], p[2] + p[3], p[4] + p[5], p[6] + p[7]]
                v = (t[0] + t[1]) + (t[2] + t[3]) + wv[r, pl.ds(IDIM, _L)]
                lvl = 0
                while stack and stack[-1][0] == lvl:
                    _lvl, a = stack.pop()
                    mk, pm = masks[lvl], perms[lvl]
                    v = jnp.where(mk, a + jnp.take(a, pm),
                                  v + jnp.take(v, pm))
                    lvl += 1
                stack.append((lvl, v))
            half = stack[0][1]
            mk, pm = masks[3], perms[3]
            full = jnp.where(mk, prev + jnp.take(prev, pm),
                             half + jnp.take(half, pm))

            @pl.when(hg % 2 == 1)
            def _():
                pmt_v[pl.ds(s * _STEP + (hg - 1) * 8, _L)] = full

            return half

        lax.fori_loop(0, _STEP // 8, group,
                      jnp.zeros((_L,), jnp.float32))

    start(0, 0)

    def ring(g, _):
        s0 = 2 * g
        start(s0 + 1, 1)
        wait_and_compute(s0, 0)

        @pl.when(s0 + 2 < _NSTEPS)
        def _():
            start(s0 + 2, 0)

        wait_and_compute(s0 + 1, 1)
        return 0

    lax.fori_loop(0, _NSTEPS // 2, ring, 0)

    for s in range(_CHUNK // IDIM):
        pltpu.sync_copy(pmt_v.at[pl.ds(s * IDIM, IDIM)],
                        pmt_hbm.at[base // IDIM + s])

    @pl.when(wid == 0)
    def _():
        pltpu.sync_copy(noise_hbm, nidx)
        pltpu.async_copy(tab_hbm.at[nidx], wnv, sem).wait()
        pltpu.sync_copy(wnv, wn_hbm)


_sc_call = functools.partial(
    pl.kernel,
    mesh=plsc.VectorSubcoreMesh(core_axis_name="c", subcore_axis_name="s"),
    compiler_params=pltpu.CompilerParams(use_tc_tiling_on_sc=False),
    out_type=[
        jax.ShapeDtypeStruct((_NR, IDIM), jnp.float32),
        jax.ShapeDtypeStruct((K, TW), jnp.float32),
    ],
    scratch_types=[
        pltpu.VMEM((_STEP, IDIM), jnp.float32),
        pltpu.VMEM((_STEP, IDIM), jnp.float32),
        pltpu.VMEM((_STEP, TW), jnp.float32),
        pltpu.VMEM((_STEP, TW), jnp.float32),
        pltpu.VMEM((_STEP,), jnp.int32),
        pltpu.VMEM((_STEP,), jnp.int32),
        pltpu.VMEM((_CHUNK,), jnp.float32),
        pltpu.VMEM((K,), jnp.int32),
        pltpu.VMEM((K, TW), jnp.float32),
        pltpu.SemaphoreType.DMA,
        pltpu.SemaphoreType.DMA,
        pltpu.SemaphoreType.DMA,
        pltpu.SemaphoreType.DMA,
        pltpu.SemaphoreType.DMA,
    ],
)(_sc_body)


_B = 1024


def _tc_body(x_ref, pmt_ref, wt_ref, cpad_ref, logits_ref):
    m = lax.dot_general(x_ref[...], wt_ref[...],
                        dimension_numbers=(((1,), (0,)), ((), ())),
                        preferred_element_type=jnp.float32)
    col = lax.broadcasted_iota(jnp.int32, (_B, KP1), 1)
    row_i = lax.broadcasted_iota(jnp.int32, (IDIM, IDIM), 0)
    col_i = lax.broadcasted_iota(jnp.int32, (IDIM, IDIM), 1)
    ident = (row_i == col_i).astype(jnp.float32)
    pm = jnp.concatenate(
        [lax.dot_general(ident, pmt_ref[pl.ds(c, 1), :],
                         dimension_numbers=(((1,), (1,)), ((), ())),
                         preferred_element_type=jnp.float32)
         for c in range(_B // IDIM)], axis=0)
    logits_ref[...] = jnp.where(col == 0, pm, -(m + cpad_ref[...]))


_tc_call = pl.pallas_call(
    _tc_body,
    grid=(N // _B,),
    in_specs=[
        pl.BlockSpec((_B, IDIM), lambda i: (i, 0)),
        pl.BlockSpec((_B // IDIM, IDIM), lambda i: (i, 0)),
        pl.BlockSpec((IDIM, KP1), lambda i: (0, 0)),
        pl.BlockSpec((1, KP1), lambda i: (0, 0)),
    ],
    out_specs=pl.BlockSpec((_B, KP1), lambda i: (i, 0)),
    out_shape=jax.ShapeDtypeStruct((N, KP1), jnp.float32),
)


import numpy as _np

_NOISE = _np.asarray(
    jax.random.randint(jax.random.key(42), (K,), 0, ODIM), dtype=_np.int32)


def kernel(input, target, weight, bias, unigram_prob):
    noise = jnp.asarray(_NOISE)
    cvec = bias - jnp.log(K * unigram_prob)
    table = jnp.concatenate(
        [weight, cvec[:, None],
         jnp.zeros((ODIM, TW - IDIM - 1), jnp.float32)], axis=1)

    pmt, wn2 = _sc_call(input, target.astype(jnp.int32), table,
                        noise.astype(jnp.int32))

    wt_pad = jnp.concatenate(
        [jnp.zeros((IDIM, 1), jnp.float32), wn2[:, :IDIM].T], axis=1)
    cn_pad = jnp.concatenate(
        [jnp.zeros((1,), jnp.float32), wn2[:, IDIM]]).reshape(1, KP1)

    logits = _tc_call(input, pmt, wt_pad, cn_pad)
    nce_target = jnp.ones((N, KP1), jnp.float32)
    return (logits, nce_target)


# speedup vs baseline: 1.1432x; 1.0703x over previous
import functools

import jax
import jax.numpy as jnp
from jax import lax
from jax.experimental import pallas as pl
from jax.experimental.pallas import tpu as pltpu
from jax.experimental.pallas import tpu_sc as plsc

N = 16384
IDIM = 128
ODIM = 1000
K = 64
KP1 = K + 1
TW = IDIM + 16

_info = plsc.get_sparse_core_info()
_NC, _NS, _L = _info.num_cores, _info.num_subcores, _info.num_lanes
_NW = _NC * _NS
_CHUNK = N // _NW
_STEP = 128
_NSTEPS = _CHUNK // _STEP
_NR = N // IDIM


def _sc_body(x_hbm, tgt_hbm, tab_hbm, noise_hbm,
             pmt_hbm, wn_hbm,
             xv0, xv1, wv0, wv1, idx0, idx1, pmt_v, nidx, wnv,
             sx0, sx1, sw0, sw1, sem):
    cid = lax.axis_index("c")
    sid = lax.axis_index("s")
    wid = sid * _NC + cid
    base = wid * _CHUNK

    xvs, wvs, idxs = (xv0, xv1), (wv0, wv1), (idx0, idx1)
    sxs, sws = (sx0, sx1), (sw0, sw1)

    lane = lax.iota(jnp.int32, _L)
    perms = [lane ^ 1, lane ^ 2, lane ^ 4, lane ^ 8]
    masks = [(lane % (2 * m)) < m for m in (1, 2, 4, 8)]

    def start(step, buf):
        rb = base + step * _STEP
        pltpu.sync_copy(tgt_hbm.at[pl.ds(rb, _STEP)], idxs[buf])
        pltpu.async_copy(x_hbm.at[pl.ds(rb, _STEP)], xvs[buf], sxs[buf])
        pltpu.async_copy(tab_hbm.at[idxs[buf]], wvs[buf], sws[buf])

    def wait_and_compute(s, buf):
        xv, wv = xvs[buf], wvs[buf]
        pltpu.make_async_copy(
            x_hbm.at[pl.ds(0, _STEP)], xv, sxs[buf]).wait()
        pltpu.make_async_copy(
            tab_hbm.at[idxs[buf]], wv, sws[buf]).wait()

        def group(hg, prev):
            stack = []
            for u in range(8):
                r = hg * 8 + u
                p = [xv[r, pl.ds(q * _L, _L)] * wv[r, pl.ds(q * _L, _L)]
                     for q in range(IDIM // _L)]
                t = [p[0] + p[1], p[2] + p[3], p[4] + p[5], p[6] + p[7]]
                v = (t[0] + t[1]) + (t[2] + t[3]) + wv[r, pl.ds(IDIM, _L)]
                lvl = 0
                while stack and stack[-1][0] == lvl:
                    _lvl, a = stack.pop()
                    mk, pm = masks[lvl], perms[lvl]
                    v = jnp.where(mk, a + jnp.take(a, pm),
                                  v + jnp.take(v, pm))
                    lvl += 1
                stack.append((lvl, v))
            half = stack[0][1]
            mk, pm = masks[3], perms[3]
            full = jnp.where(mk, prev + jnp.take(prev, pm),
                             half + jnp.take(half, pm))

            @pl.when(hg % 2 == 1)
            def _():
                pmt_v[pl.ds(s * _STEP + (hg - 1) * 8, _L)] = full

            return half

        lax.fori_loop(0, _STEP // 8, group,
                      jnp.zeros((_L,), jnp.float32))

    start(0, 0)

    def ring(g, _):
        s0 = 2 * g
        start(s0 + 1, 1)
        wait_and_compute(s0, 0)

        @pl.when(s0 + 2 < _NSTEPS)
        def _():
            start(s0 + 2, 0)

        wait_and_compute(s0 + 1, 1)
        return 0

    lax.fori_loop(0, _NSTEPS // 2, ring, 0)

    for s in range(_CHUNK // IDIM):
        pltpu.sync_copy(pmt_v.at[pl.ds(s * IDIM, IDIM)],
                        pmt_hbm.at[base // IDIM + s])

    @pl.when(wid == 0)
    def _():
        pltpu.sync_copy(noise_hbm, nidx)
        pltpu.async_copy(tab_hbm.at[nidx], wnv, sem).wait()
        pltpu.sync_copy(wnv, wn_hbm)


_sc_call = functools.partial(
    pl.kernel,
    mesh=plsc.VectorSubcoreMesh(core_axis_name="c", subcore_axis_name="s"),
    compiler_params=pltpu.CompilerParams(use_tc_tiling_on_sc=False),
    out_type=[
        jax.ShapeDtypeStruct((_NR, IDIM), jnp.float32),
        jax.ShapeDtypeStruct((K, TW), jnp.float32),
    ],
    scratch_types=[
        pltpu.VMEM((_STEP, IDIM), jnp.float32),
        pltpu.VMEM((_STEP, IDIM), jnp.float32),
        pltpu.VMEM((_STEP, TW), jnp.float32),
        pltpu.VMEM((_STEP, TW), jnp.float32),
        pltpu.VMEM((_STEP,), jnp.int32),
        pltpu.VMEM((_STEP,), jnp.int32),
        pltpu.VMEM((_CHUNK,), jnp.float32),
        pltpu.VMEM((K,), jnp.int32),
        pltpu.VMEM((K, TW), jnp.float32),
        pltpu.SemaphoreType.DMA,
        pltpu.SemaphoreType.DMA,
        pltpu.SemaphoreType.DMA,
        pltpu.SemaphoreType.DMA,
        pltpu.SemaphoreType.DMA,
    ],
)(_sc_body)


_B = 2048


def _tc_body(x_ref, pmt_ref, wt_ref, cpad_ref, logits_ref):
    m = lax.dot_general(x_ref[...], wt_ref[...],
                        dimension_numbers=(((1,), (0,)), ((), ())),
                        preferred_element_type=jnp.float32)
    col = lax.broadcasted_iota(jnp.int32, (_B, KP1), 1)
    row_i = lax.broadcasted_iota(jnp.int32, (IDIM, IDIM), 0)
    col_i = lax.broadcasted_iota(jnp.int32, (IDIM, IDIM), 1)
    ident = (row_i == col_i).astype(jnp.float32)
    pm = jnp.concatenate(
        [lax.dot_general(ident, pmt_ref[pl.ds(c, 1), :],
                         dimension_numbers=(((1,), (1,)), ((), ())),
                         preferred_element_type=jnp.float32)
         for c in range(_B // IDIM)], axis=0)
    logits_ref[...] = jnp.where(col == 0, pm, -(m + cpad_ref[...]))


_tc_call = pl.pallas_call(
    _tc_body,
    grid=(N // _B,),
    in_specs=[
        pl.BlockSpec((_B, IDIM), lambda i: (i, 0)),
        pl.BlockSpec((_B // IDIM, IDIM), lambda i: (i, 0)),
        pl.BlockSpec((IDIM, KP1), lambda i: (0, 0)),
        pl.BlockSpec((1, KP1), lambda i: (0, 0)),
    ],
    out_specs=pl.BlockSpec((_B, KP1), lambda i: (i, 0)),
    out_shape=jax.ShapeDtypeStruct((N, KP1), jnp.float32),
)


import numpy as _np

_NOISE = _np.asarray(
    jax.random.randint(jax.random.key(42), (K,), 0, ODIM), dtype=_np.int32)


def kernel(input, target, weight, bias, unigram_prob):
    noise = jnp.asarray(_NOISE)
    cvec = bias - jnp.log(K * unigram_prob)
    table = jnp.concatenate(
        [weight, cvec[:, None],
         jnp.zeros((ODIM, TW - IDIM - 1), jnp.float32)], axis=1)

    pmt, wn2 = _sc_call(input, target.astype(jnp.int32), table,
                        noise.astype(jnp.int32))

    wt_pad = jnp.concatenate(
        [jnp.zeros((IDIM, 1), jnp.float32), wn2[:, :IDIM].T], axis=1)
    cn_pad = jnp.concatenate(
        [jnp.zeros((1,), jnp.float32), wn2[:, IDIM]]).reshape(1, KP1)

    logits = _tc_call(input, pmt, wt_pad, cn_pad)
    nce_target = jnp.ones((N, KP1), jnp.float32)
    return (logits, nce_target)


# speedup vs baseline: 1.1789x; 1.0313x over previous
import functools

import jax
import jax.numpy as jnp
from jax import lax
from jax.experimental import pallas as pl
from jax.experimental.pallas import tpu as pltpu
from jax.experimental.pallas import tpu_sc as plsc

N = 16384
IDIM = 128
ODIM = 1000
K = 64
KP1 = K + 1
TW = IDIM + 16

_info = plsc.get_sparse_core_info()
_NC, _NS, _L = _info.num_cores, _info.num_subcores, _info.num_lanes
_NW = _NC * _NS
_CHUNK = N // _NW
_STEP = 128
_NSTEPS = _CHUNK // _STEP
_NR = N // IDIM


def _sc_body(x_hbm, tgt_hbm, tab_hbm, noise_hbm,
             pmt_hbm, wn_hbm,
             xv0, xv1, wv0, wv1, idx0, idx1, pmt_v, nidx, wnv,
             sx0, sx1, sw0, sw1, sem):
    cid = lax.axis_index("c")
    sid = lax.axis_index("s")
    wid = sid * _NC + cid
    base = wid * _CHUNK

    xvs, wvs, idxs = (xv0, xv1), (wv0, wv1), (idx0, idx1)
    sxs, sws = (sx0, sx1), (sw0, sw1)

    lane = lax.iota(jnp.int32, _L)
    perms = [lane ^ 1, lane ^ 2, lane ^ 4, lane ^ 8]
    masks = [(lane % (2 * m)) < m for m in (1, 2, 4, 8)]

    def start(step, buf):
        rb = base + step * _STEP
        pltpu.sync_copy(tgt_hbm.at[pl.ds(rb, _STEP)], idxs[buf])
        pltpu.async_copy(x_hbm.at[pl.ds(rb, _STEP)], xvs[buf], sxs[buf])
        pltpu.async_copy(tab_hbm.at[idxs[buf]], wvs[buf], sws[buf])

    def wait_and_compute(s, buf):
        xv, wv = xvs[buf], wvs[buf]
        pltpu.make_async_copy(
            x_hbm.at[pl.ds(0, _STEP)], xv, sxs[buf]).wait()
        pltpu.make_async_copy(
            tab_hbm.at[idxs[buf]], wv, sws[buf]).wait()

        def group(hg, prev):
            stack = []
            for u in range(8):
                r = hg * 8 + u
                p = [xv[r, pl.ds(q * _L, _L)] * wv[r, pl.ds(q * _L, _L)]
                     for q in range(IDIM // _L)]
                t = [p[0] + p[1], p[2] + p[3], p[4] + p[5], p[6] + p[7]]
                v = (t[0] + t[1]) + (t[2] + t[3]) + wv[r, pl.ds(IDIM, _L)]
                lvl = 0
                while stack and stack[-1][0] == lvl:
                    _lvl, a = stack.pop()
                    mk, pm = masks[lvl], perms[lvl]
                    v = jnp.where(mk, a + jnp.take(a, pm),
                                  v + jnp.take(v, pm))
                    lvl += 1
                stack.append((lvl, v))
            half = stack[0][1]
            mk, pm = masks[3], perms[3]
            full = jnp.where(mk, prev + jnp.take(prev, pm),
                             half + jnp.take(half, pm))

            @pl.when(hg % 2 == 1)
            def _():
                pmt_v[pl.ds(s * _STEP + (hg - 1) * 8, _L)] = full

            return half

        lax.fori_loop(0, _STEP // 8, group,
                      jnp.zeros((_L,), jnp.float32))

    start(0, 0)

    def ring(g, _):
        s0 = 2 * g
        start(s0 + 1, 1)
        wait_and_compute(s0, 0)

        @pl.when(s0 + 2 < _NSTEPS)
        def _():
            start(s0 + 2, 0)

        wait_and_compute(s0 + 1, 1)
        return 0

    lax.fori_loop(0, _NSTEPS // 2, ring, 0)

    for s in range(_CHUNK // IDIM):
        pltpu.sync_copy(pmt_v.at[pl.ds(s * IDIM, IDIM)],
                        pmt_hbm.at[base // IDIM + s])

    @pl.when(wid == 0)
    def _():
        pltpu.sync_copy(noise_hbm, nidx)
        pltpu.async_copy(tab_hbm.at[nidx], wnv, sem).wait()
        pltpu.sync_copy(wnv, wn_hbm)


_sc_call = functools.partial(
    pl.kernel,
    mesh=plsc.VectorSubcoreMesh(core_axis_name="c", subcore_axis_name="s"),
    compiler_params=pltpu.CompilerParams(use_tc_tiling_on_sc=False),
    out_type=[
        jax.ShapeDtypeStruct((_NR, IDIM), jnp.float32),
        jax.ShapeDtypeStruct((K, TW), jnp.float32),
    ],
    scratch_types=[
        pltpu.VMEM((_STEP, IDIM), jnp.float32),
        pltpu.VMEM((_STEP, IDIM), jnp.float32),
        pltpu.VMEM((_STEP, TW), jnp.float32),
        pltpu.VMEM((_STEP, TW), jnp.float32),
        pltpu.VMEM((_STEP,), jnp.int32),
        pltpu.VMEM((_STEP,), jnp.int32),
        pltpu.VMEM((_CHUNK,), jnp.float32),
        pltpu.VMEM((K,), jnp.int32),
        pltpu.VMEM((K, TW), jnp.float32),
        pltpu.SemaphoreType.DMA,
        pltpu.SemaphoreType.DMA,
        pltpu.SemaphoreType.DMA,
        pltpu.SemaphoreType.DMA,
        pltpu.SemaphoreType.DMA,
    ],
)(_sc_body)


_B = 4096


def _tc_body(x_ref, pmt_ref, wt_ref, cpad_ref, logits_ref):
    m = lax.dot_general(x_ref[...], wt_ref[...],
                        dimension_numbers=(((1,), (0,)), ((), ())),
                        preferred_element_type=jnp.float32)
    col = lax.broadcasted_iota(jnp.int32, (_B, KP1), 1)
    row_i = lax.broadcasted_iota(jnp.int32, (IDIM, IDIM), 0)
    col_i = lax.broadcasted_iota(jnp.int32, (IDIM, IDIM), 1)
    ident = (row_i == col_i).astype(jnp.float32)
    pm = jnp.concatenate(
        [lax.dot_general(ident, pmt_ref[pl.ds(c, 1), :],
                         dimension_numbers=(((1,), (1,)), ((), ())),
                         preferred_element_type=jnp.float32)
         for c in range(_B // IDIM)], axis=0)
    logits_ref[...] = jnp.where(col == 0, pm, -(m + cpad_ref[...]))


_tc_call = pl.pallas_call(
    _tc_body,
    grid=(N // _B,),
    in_specs=[
        pl.BlockSpec((_B, IDIM), lambda i: (i, 0)),
        pl.BlockSpec((_B // IDIM, IDIM), lambda i: (i, 0)),
        pl.BlockSpec((IDIM, KP1), lambda i: (0, 0)),
        pl.BlockSpec((1, KP1), lambda i: (0, 0)),
    ],
    out_specs=pl.BlockSpec((_B, KP1), lambda i: (i, 0)),
    out_shape=jax.ShapeDtypeStruct((N, KP1), jnp.float32),
)


import numpy as _np

_NOISE = _np.asarray(
    jax.random.randint(jax.random.key(42), (K,), 0, ODIM), dtype=_np.int32)


def kernel(input, target, weight, bias, unigram_prob):
    noise = jnp.asarray(_NOISE)
    cvec = bias - jnp.log(K * unigram_prob)
    table = jnp.concatenate(
        [weight, cvec[:, None],
         jnp.zeros((ODIM, TW - IDIM - 1), jnp.float32)], axis=1)

    pmt, wn2 = _sc_call(input, target.astype(jnp.int32), table,
                        noise.astype(jnp.int32))

    wt_pad = jnp.concatenate(
        [jnp.zeros((IDIM, 1), jnp.float32), wn2[:, :IDIM].T], axis=1)
    cn_pad = jnp.concatenate(
        [jnp.zeros((1,), jnp.float32), wn2[:, IDIM]]).reshape(1, KP1)

    logits = _tc_call(input, pmt, wt_pad, cn_pad)
    nce_target = jnp.ones((N, KP1), jnp.float32)
    return (logits, nce_target)
